# baseline (speedup 1.0000x reference)
import functools

import numpy as np
import jax
import jax.numpy as jnp
from jax import lax
from jax.experimental import pallas as pl
from jax.experimental.pallas import tpu as pltpu
from jax.experimental.pallas import tpu_sc as plsc

ROWS = 64
COLS = 32768
LANES = 16
NUM_CORES = 2
NUM_SUBCORES = 16
NUM_WORKERS = NUM_CORES * NUM_SUBCORES
ROWS_PER_WORKER = ROWS // NUM_WORKERS
CHUNKS = COLS // LANES
UNROLL = 8
BIG = 2**30


def _row_argmax(row_ref):
  lane_iota = lax.iota(jnp.int32, LANES)

  init_max = []
  init_idx = []
  for u in range(UNROLL):
    v = row_ref[pl.ds(u * LANES, LANES)]
    init_max.append(v)
    init_idx.append(lane_iota + u * LANES)

  def body(g, carry):
    maxs = list(carry[0])
    idxs = list(carry[1])
    base = g * (UNROLL * LANES)
    for u in range(UNROLL):
      v = row_ref[pl.ds(base + u * LANES, LANES)]
      cand_idx = lane_iota + (base + u * LANES)
      gt = v > maxs[u]
      maxs[u] = jnp.where(gt, v, maxs[u])
      idxs[u] = jnp.where(gt, cand_idx, idxs[u])
    return (tuple(maxs), tuple(idxs))

  maxs, idxs = lax.fori_loop(
      1, CHUNKS // UNROLL, body, (tuple(init_max), tuple(init_idx)))

  gmax_v = maxs[0]
  for u in range(1, UNROLL):
    gmax_v = jnp.maximum(gmax_v, maxs[u])
  gmax = jnp.max(gmax_v, axis=0)
  big_v = jnp.full((LANES,), BIG, jnp.int32)
  best = big_v
  for u in range(UNROLL):
    best = jnp.minimum(best, jnp.where(maxs[u] == gmax, idxs[u], big_v))
  return jnp.min(best, axis=0)


def _body(x_hbm, out_hbm, row0_v, row1_v, res_v, sem0, sem1):
  wid = lax.axis_index("s") * NUM_CORES + lax.axis_index("c")
  r0 = wid * ROWS_PER_WORKER
  cp0 = pltpu.make_async_copy(x_hbm.at[r0], row0_v, sem0)
  cp1 = pltpu.make_async_copy(x_hbm.at[r0 + 1], row1_v, sem1)
  cp0.start()
  cp1.start()
  cp0.wait()
  a0 = _row_argmax(row0_v)
  cp1.wait()
  a1 = _row_argmax(row1_v)

  lane_iota = lax.iota(jnp.int32, LANES)
  res = jnp.where(lane_iota == 0, a0, a1)
  res_v[...] = res
  pltpu.sync_copy(res_v, out_hbm.at[wid])


@jax.jit
def kernel(x):
  mesh = plsc.VectorSubcoreMesh(
      core_axis_name="c", subcore_axis_name="s",
      num_cores=NUM_CORES, num_subcores=NUM_SUBCORES)
  padded = pl.kernel(
      _body,
      out_type=jax.ShapeDtypeStruct((NUM_WORKERS, LANES), jnp.int32),
      mesh=mesh,
      scratch_types=[
          pltpu.VMEM((COLS,), jnp.float32),
          pltpu.VMEM((COLS,), jnp.float32),
          pltpu.VMEM((LANES,), jnp.int32),
          pltpu.SemaphoreType.DMA,
          pltpu.SemaphoreType.DMA,
      ],
      compiler_params=pltpu.CompilerParams(needs_layout_passes=False),
  )(x)
  return padded[:, :ROWS_PER_WORKER].reshape(ROWS)
